# Initial kernel scaffold; baseline (speedup 1.0000x reference)
import jax
import jax.numpy as jnp
from jax.experimental import pallas as pl


def kernel(node_feat_continuous, node_feat_discrete, edge_feat_continuous, edge_index, emb_node, W_node, b_node, emb_dist, W_dist, b_dist, W_edge, b_edge, W_agg, b_agg, W_b2a, b_b2a, W_m1, b_m1, W_m2, b_m2, W_out, b_out, W_sc, b_sc):
    raise NotImplementedError("write your pallas kernel here")


# speedup vs baseline: 2.2196x; 2.2196x over previous
import functools

import jax
import jax.numpy as jnp
from jax import lax
from jax.experimental import pallas as pl
from jax.experimental.pallas import tpu as pltpu
from jax.experimental.pallas import tpu_sc as plsc

N = 10000
E = 160000
HID = 128
RBF = 64
CUT = 6.0

NP = 10240
EP = 163840
NWORK = 32
CHUNK = 64
NCHUNK = EP // (NWORK * CHUNK)
EDGES_PER_W = EP // NWORK
ROWS_PER_TILE = NP // 16

BN_A = 1024
BE_B = 2048
BN_2 = 1000


def _stage_a_body(disc_ref, emb_ref, wn_ref, bn_ref, w1_ref, w2_ref,
                  a_ref, b_ref):
    disc = disc_ref[...]
    vocab_ids = lax.broadcasted_iota(jnp.int32, (1, 64), 1)
    onehot = (disc == vocab_ids).astype(jnp.float32)
    x = jnp.dot(onehot, emb_ref[...], preferred_element_type=jnp.float32, precision=lax.Precision.HIGHEST)
    h = jnp.maximum(
        jnp.dot(x, wn_ref[...], preferred_element_type=jnp.float32)
        + bn_ref[...], 0.0)
    a_ref[...] = jnp.dot(h, w1_ref[...], preferred_element_type=jnp.float32)
    b_ref[...] = jnp.dot(h, w2_ref[...], preferred_element_type=jnp.float32)


def _stage_a(disc_p, emb_node, w_node, b_node, w1, w2):
    grid = NP // BN_A
    return pl.pallas_call(
        _stage_a_body,
        grid=(grid,),
        in_specs=[
            pl.BlockSpec((BN_A, 1), lambda i: (i, 0)),
            pl.BlockSpec((64, HID), lambda i: (0, 0)),
            pl.BlockSpec((HID, HID), lambda i: (0, 0)),
            pl.BlockSpec((1, HID), lambda i: (0, 0)),
            pl.BlockSpec((HID, HID), lambda i: (0, 0)),
            pl.BlockSpec((HID, HID), lambda i: (0, 0)),
        ],
        out_specs=[
            pl.BlockSpec((BN_A, HID), lambda i: (i, 0)),
            pl.BlockSpec((BN_A, HID), lambda i: (i, 0)),
        ],
        out_shape=[
            jax.ShapeDtypeStruct((NP, HID), jnp.float32),
            jax.ShapeDtypeStruct((NP, HID), jnp.float32),
        ],
    )(disc_p, emb_node, w_node, b_node, w1, w2)


def _stage_b_body(d_ref, we_ref, be_ref, embd_ref, wd_ref, bd_ref,
                  w3_ref, bagg_ref, c_ref):
    i = pl.program_id(0)
    d = d_ref[...]
    centers = (lax.broadcasted_iota(jnp.int32, (1, RBF), 1).astype(jnp.float32)
               * (CUT / (RBF - 1)))
    diff = d - centers
    r = d * (1.0 / CUT)
    r2 = r * r
    r3 = r2 * r
    r4 = r2 * r2
    r5 = r4 * r
    env = jnp.clip(1.0 - 6.0 * r5 + 15.0 * r4 - 10.0 * r3, 0.0, 1.0)
    rbf = jnp.exp(-10.0 * diff * diff) * env
    eh_rbf = jnp.maximum(
        jnp.dot(rbf, we_ref[...], preferred_element_type=jnp.float32)
        + be_ref[...], 0.0)
    t8 = jnp.maximum(
        jnp.dot(embd_ref[...], wd_ref[...],
                preferred_element_type=jnp.float32) + bd_ref[...], 0.0)
    dist_idx = jnp.clip(d, 1.0, 4.99999).astype(jnp.int32) - 1
    slot_ids = lax.broadcasted_iota(jnp.int32, (1, 8), 1)
    oh = (dist_idx == slot_ids).astype(jnp.float32)
    eh_emb = jnp.dot(oh, t8, preferred_element_type=jnp.float32,
                     precision=lax.Precision.HIGHEST)
    edge_feat = eh_rbf + eh_emb
    c = (jnp.dot(edge_feat, w3_ref[...], preferred_element_type=jnp.float32)
         + bagg_ref[...])
    rows = i * BE_B + lax.broadcasted_iota(jnp.int32, (BE_B, 1), 0)
    c_ref[...] = jnp.where(rows < E, c, 0.0)


def _stage_b(d_p, w_edge, b_edge, emb_dist8, w_dist, b_dist, w3, b_agg):
    grid = EP // BE_B
    return pl.pallas_call(
        _stage_b_body,
        grid=(grid,),
        in_specs=[
            pl.BlockSpec((BE_B, 1), lambda i: (i, 0)),
            pl.BlockSpec((RBF, HID), lambda i: (0, 0)),
            pl.BlockSpec((1, HID), lambda i: (0, 0)),
            pl.BlockSpec((8, HID), lambda i: (0, 0)),
            pl.BlockSpec((HID, HID), lambda i: (0, 0)),
            pl.BlockSpec((1, HID), lambda i: (0, 0)),
            pl.BlockSpec((HID, HID), lambda i: (0, 0)),
            pl.BlockSpec((1, HID), lambda i: (0, 0)),
        ],
        out_specs=pl.BlockSpec((BE_B, HID), lambda i: (i, 0)),
        out_shape=jax.ShapeDtypeStruct((EP, HID), jnp.float32),
    )(d_p, w_edge, b_edge, emb_dist8, w_dist, b_dist, w3, b_agg)


def _sc_body(a_hbm, b_hbm, c_hbm, src_hbm, dst_hbm, out_hbm,
             idx_s, idx_d, arows, brows, crows, zrow, accum, sem_a, sem_b):
    cid = lax.axis_index("c")
    sid = lax.axis_index("s")
    wid = sid * 2 + cid
    base = wid * EDGES_PER_W

    def zfill(i, _):
        def zfill_inner(j, _):
            zrow[i, pl.ds(j * 16, 16)] = jnp.zeros((16,), jnp.float32)
            return 0
        return lax.fori_loop(0, HID // 16, zfill_inner, 0)
    lax.fori_loop(0, CHUNK, zfill, 0)
    for k in range(ROWS_PER_TILE // CHUNK):
        pltpu.sync_copy(zrow,
                        accum.at[pl.ds(sid * ROWS_PER_TILE + k * CHUNK, CHUNK)])
    plsc.subcore_barrier()

    def chunk_body(j, _):
        off = base + j * CHUNK
        pltpu.sync_copy(src_hbm.at[pl.ds(off, CHUNK)], idx_s)
        pltpu.sync_copy(dst_hbm.at[pl.ds(off, CHUNK)], idx_d)
        ga = pltpu.async_copy(a_hbm.at[idx_s], arows, sem_a)
        gb = pltpu.async_copy(b_hbm.at[idx_d], brows, sem_b)
        pltpu.sync_copy(c_hbm.at[pl.ds(off, CHUNK)], crows)
        ga.wait()
        gb.wait()

        def row_body(i, _):
            def lane_body(jj, _):
                sl = pl.ds(jj * 16, 16)
                v = arows[i, sl] + brows[i, sl] + crows[i, sl]
                arows[i, sl] = jnp.maximum(v, 0.0)
                return 0
            return lax.fori_loop(0, HID // 16, lane_body, 0)
        lax.fori_loop(0, CHUNK, row_body, 0)

        junk = jnp.full((16,), NP - 1, jnp.int32)
        nv = CHUNK // 16
        iv = [idx_d[pl.ds(t * 16, 16)] for t in range(nv)]
        pos = [lax.iota(jnp.int32, 16) + t * 16 for t in range(nv)]
        one = jnp.full((16,), 1, jnp.int32)
        zero = jnp.zeros((16,), jnp.int32)
        occ = [zero for _ in range(nv)]
        for j in range(CHUNK):
            t0, q = j // 16, j % 16
            b = jnp.full((16,), iv[t0][q], jnp.int32)
            for u in range(t0, nv):
                later = pos[u] > j
                if u == t0 and q == 15:
                    continue
                occ[u] = occ[u] + jnp.where((iv[u] == b) & later, one, zero)
        for u in range(nv):
            occ[u] = jnp.where(iv[u] == junk, zero, occ[u])
        mx = jnp.maximum(jnp.maximum(occ[0], occ[1]),
                         jnp.maximum(occ[2], occ[3]))
        nrounds = mx[0]
        for q in range(1, 16):
            nrounds = jnp.maximum(nrounds, mx[q])
        nrounds = nrounds + 1

        def round_body(r, _):
            rv = jnp.full((16,), r, jnp.int32)
            for u in range(nv):
                m = (occ[u] == rv) & (iv[u] != junk)
                idx_d[pl.ds(u * 16, 16)] = jnp.where(m, iv[u], junk)
            pltpu.sync_copy(arows, accum.at[idx_d], add=True)
            return 0
        lax.fori_loop(0, nrounds, round_body, 0)
        return 0

    lax.fori_loop(0, NCHUNK, chunk_body, 0)
    plsc.subcore_barrier()
    pltpu.sync_copy(accum.at[pl.ds(sid * ROWS_PER_TILE, ROWS_PER_TILE)],
                    out_hbm.at[cid, pl.ds(sid * ROWS_PER_TILE, ROWS_PER_TILE)])


def _stage_sc(a, b, c, src_p, dst_p):
    mesh = plsc.VectorSubcoreMesh(core_axis_name="c", subcore_axis_name="s")
    fn = pl.kernel(
        _sc_body,
        mesh=mesh,
        out_type=jax.ShapeDtypeStruct((2, NP, HID), jnp.float32),
        scratch_types=[
            pltpu.VMEM((CHUNK,), jnp.int32),
            pltpu.VMEM((CHUNK,), jnp.int32),
            pltpu.VMEM((CHUNK, HID), jnp.float32),
            pltpu.VMEM((CHUNK, HID), jnp.float32),
            pltpu.VMEM((CHUNK, HID), jnp.float32),
            pltpu.VMEM((CHUNK, HID), jnp.float32),
            pltpu.VMEM_SHARED((NP, HID), jnp.float32),
            pltpu.SemaphoreType.DMA,
            pltpu.SemaphoreType.DMA,
        ],
    )
    return fn(a, b, c, src_p, dst_p)


def _stage_2_body(p_ref, nc_ref, wt_ref, wb_ref, bb_ref, wm1_ref, bm1_ref,
                  wm2_ref, bm2_ref, wh_ref, bh_ref, out_ref, gacc):
    i = pl.program_id(0)

    @pl.when(i == 0)
    def _():
        gacc[...] = jnp.zeros_like(gacc)

    atom = p_ref[0] + p_ref[1]
    h2 = jnp.maximum(
        jnp.dot(atom, wt_ref[...], preferred_element_type=jnp.float32)
        + jnp.dot(nc_ref[...], wb_ref[...], preferred_element_type=jnp.float32)
        + bb_ref[...], 0.0)
    gacc[0:1, :] = gacc[0:1, :] + jnp.sum(h2, axis=0, keepdims=True)

    @pl.when(i == (N // BN_2) - 1)
    def _():
        g = gacc[0:1, :]
        m1 = jnp.maximum(
            jnp.dot(g, wm1_ref[...], preferred_element_type=jnp.float32, precision=lax.Precision.HIGHEST)
            + bm1_ref[...], 0.0)
        m2 = jnp.maximum(
            jnp.dot(m1, wm2_ref[...], preferred_element_type=jnp.float32, precision=lax.Precision.HIGHEST)
            + bm2_ref[...], 0.0)
        out_ref[...] = (jnp.dot(m2, wh_ref[...],
                                preferred_element_type=jnp.float32, precision=lax.Precision.HIGHEST)
                        + bh_ref[...])


def _stage_2(parts, ncont, wt, wb, bb, wm1, bm1, wm2, bm2, whead, bhead):
    grid = N // BN_2
    return pl.pallas_call(
        _stage_2_body,
        grid=(grid,),
        in_specs=[
            pl.BlockSpec((2, BN_2, HID), lambda i: (0, i, 0)),
            pl.BlockSpec((BN_2, HID), lambda i: (i, 0)),
            pl.BlockSpec((HID, HID), lambda i: (0, 0)),
            pl.BlockSpec((HID, HID), lambda i: (0, 0)),
            pl.BlockSpec((1, HID), lambda i: (0, 0)),
            pl.BlockSpec((HID, 256), lambda i: (0, 0)),
            pl.BlockSpec((1, 256), lambda i: (0, 0)),
            pl.BlockSpec((256, HID), lambda i: (0, 0)),
            pl.BlockSpec((1, HID), lambda i: (0, 0)),
            pl.BlockSpec((HID, 37), lambda i: (0, 0)),
            pl.BlockSpec((1, 37), lambda i: (0, 0)),
        ],
        out_specs=pl.BlockSpec((1, 37), lambda i: (0, 0)),
        out_shape=jax.ShapeDtypeStruct((1, 37), jnp.float32),
        scratch_shapes=[pltpu.VMEM((8, HID), jnp.float32)],
    )(parts, ncont, wt, wb, bb, wm1, bm1, wm2, bm2, whead, bhead)


def kernel(node_feat_continuous, node_feat_discrete, edge_feat_continuous,
           edge_index, emb_node, W_node, b_node, emb_dist, W_dist, b_dist,
           W_edge, b_edge, W_agg, b_agg, W_b2a, b_b2a, W_m1, b_m1,
           W_m2, b_m2, W_out, b_out, W_sc, b_sc):
    f32 = jnp.float32
    disc = node_feat_discrete.astype(jnp.int32).reshape(N, 1)
    disc_p = jnp.pad(disc, ((0, NP - N), (0, 0)))
    src_p = jnp.pad(edge_index[0].astype(jnp.int32), (0, EP - E),
                    constant_values=NP - 1)
    dst_p = jnp.pad(edge_index[1].astype(jnp.int32), (0, EP - E),
                    constant_values=NP - 1)
    d_p = jnp.pad(edge_feat_continuous.astype(f32), ((0, EP - E), (0, 0)))

    w1 = W_agg[:HID]
    w2 = W_agg[HID:2 * HID]
    w3 = W_agg[2 * HID:]
    emb_dist8 = jnp.pad(emb_dist.astype(f32), ((0, 3), (0, 0)))
    whead = jnp.concatenate([W_out, W_sc], axis=1)
    bhead = jnp.concatenate([b_out, b_sc]).reshape(1, 37)

    a, b = _stage_a(disc_p, emb_node.astype(f32), W_node, b_node.reshape(1, HID),
                    w1, w2)
    c = _stage_b(d_p, W_edge, b_edge.reshape(1, HID), emb_dist8, W_dist,
                 b_dist.reshape(1, HID), w3, b_agg.reshape(1, HID))
    parts = _stage_sc(a, b, c, src_p, dst_p)
    out37 = _stage_2(parts, node_feat_continuous.astype(f32),
                     W_b2a[:HID], W_b2a[HID:], b_b2a.reshape(1, HID),
                     W_m1, b_m1.reshape(1, 256), W_m2, b_m2.reshape(1, HID),
                     whead, bhead)
    return out37[:, 0:1], out37[:, 1:37]


# speedup vs baseline: 2.3522x; 1.0597x over previous
import functools

import jax
import jax.numpy as jnp
from jax import lax
from jax.experimental import pallas as pl
from jax.experimental.pallas import tpu as pltpu
from jax.experimental.pallas import tpu_sc as plsc

N = 10000
E = 160000
HID = 128
RBF = 64
CUT = 6.0

NP = 10240
EP = 163840
NWORK = 32
CHUNK = 64
NCHUNK = EP // (NWORK * CHUNK)
EDGES_PER_W = EP // NWORK
K0 = 62
K1 = 98
OFF0 = 16 * K0 * CHUNK
ROWS_PER_TILE = NP // 16

BN_A = 1024
BE_B = 2000
BN_2 = 1000


def _stage_a_body(disc_ref, emb_ref, wn_ref, bn_ref, w1_ref, w2_ref,
                  a_ref, b_ref):
    disc = disc_ref[...]
    vocab_ids = lax.broadcasted_iota(jnp.int32, (1, 64), 1)
    onehot = (disc == vocab_ids).astype(jnp.float32)
    x = jnp.dot(onehot, emb_ref[...], preferred_element_type=jnp.float32, precision=lax.Precision.HIGHEST)
    h = jnp.maximum(
        jnp.dot(x, wn_ref[...], preferred_element_type=jnp.float32)
        + bn_ref[...], 0.0)
    a_ref[...] = jnp.dot(h, w1_ref[...], preferred_element_type=jnp.float32)
    b_ref[...] = jnp.dot(h, w2_ref[...], preferred_element_type=jnp.float32)


def _stage_a(disc_p, emb_node, w_node, b_node, w1, w2):
    grid = NP // BN_A
    return pl.pallas_call(
        _stage_a_body,
        grid=(grid,),
        in_specs=[
            pl.BlockSpec((BN_A, 1), lambda i: (i, 0)),
            pl.BlockSpec((64, HID), lambda i: (0, 0)),
            pl.BlockSpec((HID, HID), lambda i: (0, 0)),
            pl.BlockSpec((1, HID), lambda i: (0, 0)),
            pl.BlockSpec((HID, HID), lambda i: (0, 0)),
            pl.BlockSpec((HID, HID), lambda i: (0, 0)),
        ],
        out_specs=[
            pl.BlockSpec((BN_A, HID), lambda i: (i, 0)),
            pl.BlockSpec((BN_A, HID), lambda i: (i, 0)),
        ],
        out_shape=[
            jax.ShapeDtypeStruct((NP, HID), jnp.float32),
            jax.ShapeDtypeStruct((NP, HID), jnp.float32),
        ],
    )(disc_p, emb_node, w_node, b_node, w1, w2)


def _stage_b_body(d_ref, we_ref, be_ref, embd_ref, wd_ref, bd_ref,
                  w3_ref, bagg_ref, c_ref):
    d = d_ref[...]
    centers = (lax.broadcasted_iota(jnp.int32, (1, RBF), 1).astype(jnp.float32)
               * (CUT / (RBF - 1)))
    diff = d - centers
    r = d * (1.0 / CUT)
    r2 = r * r
    r3 = r2 * r
    r4 = r2 * r2
    r5 = r4 * r
    env = jnp.clip(1.0 - 6.0 * r5 + 15.0 * r4 - 10.0 * r3, 0.0, 1.0)
    rbf = jnp.exp(-10.0 * diff * diff) * env
    eh_rbf = jnp.maximum(
        jnp.dot(rbf, we_ref[...], preferred_element_type=jnp.float32)
        + be_ref[...], 0.0)
    t8 = jnp.maximum(
        jnp.dot(embd_ref[...], wd_ref[...],
                preferred_element_type=jnp.float32) + bd_ref[...], 0.0)
    dist_idx = jnp.clip(d, 1.0, 4.99999).astype(jnp.int32) - 1
    slot_ids = lax.broadcasted_iota(jnp.int32, (1, 8), 1)
    oh = (dist_idx == slot_ids).astype(jnp.float32)
    eh_emb = jnp.dot(oh, t8, preferred_element_type=jnp.float32,
                     precision=lax.Precision.HIGHEST)
    edge_feat = eh_rbf + eh_emb
    c_ref[...] = (jnp.dot(edge_feat, w3_ref[...],
                          preferred_element_type=jnp.float32)
                  + bagg_ref[...])


def _stage_b(d_p, w_edge, b_edge, emb_dist8, w_dist, b_dist, w3, b_agg):
    grid = E // BE_B
    return pl.pallas_call(
        _stage_b_body,
        grid=(grid,),
        in_specs=[
            pl.BlockSpec((BE_B, 1), lambda i: (i, 0)),
            pl.BlockSpec((RBF, HID), lambda i: (0, 0)),
            pl.BlockSpec((1, HID), lambda i: (0, 0)),
            pl.BlockSpec((8, HID), lambda i: (0, 0)),
            pl.BlockSpec((HID, HID), lambda i: (0, 0)),
            pl.BlockSpec((1, HID), lambda i: (0, 0)),
            pl.BlockSpec((HID, HID), lambda i: (0, 0)),
            pl.BlockSpec((1, HID), lambda i: (0, 0)),
        ],
        out_specs=pl.BlockSpec((BE_B, HID), lambda i: (i, 0)),
        out_shape=jax.ShapeDtypeStruct((EP, HID), jnp.float32),
    )(d_p, w_edge, b_edge, emb_dist8, w_dist, b_dist, w3, b_agg)


def _sc_body(a_hbm, b_hbm, c_hbm, src_hbm, dst_hbm, out_hbm,
             idx_s, idx_d, arows, brows, crows, zrow, accum, sem_a, sem_b):
    cid = lax.axis_index("c")
    sid = lax.axis_index("s")
    nchunks = jnp.where(cid == 0, K0, K1)
    base = jnp.where(cid == 0, sid * (K0 * CHUNK), OFF0 + sid * (K1 * CHUNK))

    def zfill(i, _):
        def zfill_inner(j, _):
            zrow[i, pl.ds(j * 16, 16)] = jnp.zeros((16,), jnp.float32)
            return 0
        return lax.fori_loop(0, HID // 16, zfill_inner, 0)
    lax.fori_loop(0, CHUNK, zfill, 0)
    for k in range(ROWS_PER_TILE // CHUNK):
        pltpu.sync_copy(zrow,
                        accum.at[pl.ds(sid * ROWS_PER_TILE + k * CHUNK, CHUNK)])
    plsc.subcore_barrier()

    def chunk_body(j, _):
        off = base + j * CHUNK
        pltpu.sync_copy(src_hbm.at[pl.ds(off, CHUNK)], idx_s)
        pltpu.sync_copy(dst_hbm.at[pl.ds(off, CHUNK)], idx_d)
        ga = pltpu.async_copy(a_hbm.at[idx_s], arows, sem_a)
        gb = pltpu.async_copy(b_hbm.at[idx_d], brows, sem_b)
        pltpu.sync_copy(c_hbm.at[pl.ds(off, CHUNK)], crows)
        ga.wait()
        gb.wait()

        def row_body(i, _):
            def lane_body(jj, _):
                sl = pl.ds(jj * 16, 16)
                v = arows[i, sl] + brows[i, sl] + crows[i, sl]
                arows[i, sl] = jnp.maximum(v, 0.0)
                return 0
            return lax.fori_loop(0, HID // 16, lane_body, 0)
        lax.fori_loop(0, CHUNK, row_body, 0)

        junk = jnp.full((16,), NP - 1, jnp.int32)
        nv = CHUNK // 16
        iv = [idx_d[pl.ds(t * 16, 16)] for t in range(nv)]
        pos = [lax.iota(jnp.int32, 16) + t * 16 for t in range(nv)]
        one = jnp.full((16,), 1, jnp.int32)
        zero = jnp.zeros((16,), jnp.int32)
        occ = [zero for _ in range(nv)]
        for j in range(CHUNK):
            t0, q = j // 16, j % 16
            b = jnp.full((16,), iv[t0][q], jnp.int32)
            for u in range(t0, nv):
                later = pos[u] > j
                if u == t0 and q == 15:
                    continue
                occ[u] = occ[u] + jnp.where((iv[u] == b) & later, one, zero)
        for u in range(nv):
            occ[u] = jnp.where(iv[u] == junk, zero, occ[u])
        mx = jnp.maximum(jnp.maximum(occ[0], occ[1]),
                         jnp.maximum(occ[2], occ[3]))
        nrounds = mx[0]
        for q in range(1, 16):
            nrounds = jnp.maximum(nrounds, mx[q])
        nrounds = nrounds + 1

        def round_body(r, _):
            rv = jnp.full((16,), r, jnp.int32)
            for u in range(nv):
                m = (occ[u] == rv) & (iv[u] != junk)
                idx_d[pl.ds(u * 16, 16)] = jnp.where(m, iv[u], junk)
            pltpu.sync_copy(arows, accum.at[idx_d], add=True)
            return 0
        lax.fori_loop(0, nrounds, round_body, 0)
        return 0

    lax.fori_loop(0, nchunks, chunk_body, 0)
    plsc.subcore_barrier()
    pltpu.sync_copy(accum.at[pl.ds(sid * ROWS_PER_TILE, ROWS_PER_TILE)],
                    out_hbm.at[cid, pl.ds(sid * ROWS_PER_TILE, ROWS_PER_TILE)])


def _stage_sc(a, b, c, src_p, dst_p):
    mesh = plsc.VectorSubcoreMesh(core_axis_name="c", subcore_axis_name="s")
    fn = pl.kernel(
        _sc_body,
        mesh=mesh,
        out_type=jax.ShapeDtypeStruct((2, NP, HID), jnp.float32),
        scratch_types=[
            pltpu.VMEM((CHUNK,), jnp.int32),
            pltpu.VMEM((CHUNK,), jnp.int32),
            pltpu.VMEM((CHUNK, HID), jnp.float32),
            pltpu.VMEM((CHUNK, HID), jnp.float32),
            pltpu.VMEM((CHUNK, HID), jnp.float32),
            pltpu.VMEM((CHUNK, HID), jnp.float32),
            pltpu.VMEM_SHARED((NP, HID), jnp.float32),
            pltpu.SemaphoreType.DMA,
            pltpu.SemaphoreType.DMA,
        ],
    )
    return fn(a, b, c, src_p, dst_p)


def _stage_2_body(p_ref, nc_ref, wt_ref, wb_ref, bb_ref, wm1_ref, bm1_ref,
                  wm2_ref, bm2_ref, wh_ref, bh_ref, out_ref, gacc):
    i = pl.program_id(0)

    @pl.when(i == 0)
    def _():
        gacc[...] = jnp.zeros_like(gacc)

    atom = p_ref[0] + p_ref[1]
    h2 = jnp.maximum(
        jnp.dot(atom, wt_ref[...], preferred_element_type=jnp.float32)
        + jnp.dot(nc_ref[...], wb_ref[...], preferred_element_type=jnp.float32)
        + bb_ref[...], 0.0)
    gacc[0:1, :] = gacc[0:1, :] + jnp.sum(h2, axis=0, keepdims=True)

    @pl.when(i == (N // BN_2) - 1)
    def _():
        g = gacc[0:1, :]
        m1 = jnp.maximum(
            jnp.dot(g, wm1_ref[...], preferred_element_type=jnp.float32, precision=lax.Precision.HIGHEST)
            + bm1_ref[...], 0.0)
        m2 = jnp.maximum(
            jnp.dot(m1, wm2_ref[...], preferred_element_type=jnp.float32, precision=lax.Precision.HIGHEST)
            + bm2_ref[...], 0.0)
        out_ref[...] = (jnp.dot(m2, wh_ref[...],
                                preferred_element_type=jnp.float32, precision=lax.Precision.HIGHEST)
                        + bh_ref[...])


def _stage_2(parts, ncont, wt, wb, bb, wm1, bm1, wm2, bm2, whead, bhead):
    grid = N // BN_2
    return pl.pallas_call(
        _stage_2_body,
        grid=(grid,),
        in_specs=[
            pl.BlockSpec((2, BN_2, HID), lambda i: (0, i, 0)),
            pl.BlockSpec((BN_2, HID), lambda i: (i, 0)),
            pl.BlockSpec((HID, HID), lambda i: (0, 0)),
            pl.BlockSpec((HID, HID), lambda i: (0, 0)),
            pl.BlockSpec((1, HID), lambda i: (0, 0)),
            pl.BlockSpec((HID, 256), lambda i: (0, 0)),
            pl.BlockSpec((1, 256), lambda i: (0, 0)),
            pl.BlockSpec((256, HID), lambda i: (0, 0)),
            pl.BlockSpec((1, HID), lambda i: (0, 0)),
            pl.BlockSpec((HID, 37), lambda i: (0, 0)),
            pl.BlockSpec((1, 37), lambda i: (0, 0)),
        ],
        out_specs=pl.BlockSpec((1, 37), lambda i: (0, 0)),
        out_shape=jax.ShapeDtypeStruct((1, 37), jnp.float32),
        scratch_shapes=[pltpu.VMEM((8, HID), jnp.float32)],
    )(parts, ncont, wt, wb, bb, wm1, bm1, wm2, bm2, whead, bhead)


def kernel(node_feat_continuous, node_feat_discrete, edge_feat_continuous,
           edge_index, emb_node, W_node, b_node, emb_dist, W_dist, b_dist,
           W_edge, b_edge, W_agg, b_agg, W_b2a, b_b2a, W_m1, b_m1,
           W_m2, b_m2, W_out, b_out, W_sc, b_sc):
    f32 = jnp.float32
    disc = node_feat_discrete.astype(jnp.int32).reshape(N, 1)
    disc_p = jnp.pad(disc, ((0, NP - N), (0, 0)))
    src_p = jnp.pad(edge_index[0].astype(jnp.int32), (0, EP - E),
                    constant_values=NP - 1)
    dst_p = jnp.pad(edge_index[1].astype(jnp.int32), (0, EP - E),
                    constant_values=NP - 1)
    d_p = edge_feat_continuous.astype(f32)

    w1 = W_agg[:HID]
    w2 = W_agg[HID:2 * HID]
    w3 = W_agg[2 * HID:]
    emb_dist8 = jnp.pad(emb_dist.astype(f32), ((0, 3), (0, 0)))
    whead = jnp.concatenate([W_out, W_sc], axis=1)
    bhead = jnp.concatenate([b_out, b_sc]).reshape(1, 37)

    a, b = _stage_a(disc_p, emb_node.astype(f32), W_node, b_node.reshape(1, HID),
                    w1, w2)
    c = _stage_b(d_p, W_edge, b_edge.reshape(1, HID), emb_dist8, W_dist,
                 b_dist.reshape(1, HID), w3, b_agg.reshape(1, HID))
    parts = _stage_sc(a, b, c, src_p, dst_p)
    out37 = _stage_2(parts, node_feat_continuous.astype(f32),
                     W_b2a[:HID], W_b2a[HID:], b_b2a.reshape(1, HID),
                     W_m1, b_m1.reshape(1, 256), W_m2, b_m2.reshape(1, HID),
                     whead, bhead)
    return out37[:, 0:1], out37[:, 1:37]


# speedup vs baseline: 2.6921x; 1.1445x over previous
import functools

import jax
import jax.numpy as jnp
from jax import lax
from jax.experimental import pallas as pl
from jax.experimental.pallas import tpu as pltpu
from jax.experimental.pallas import tpu_sc as plsc

N = 10000
E = 160000
HID = 128
RBF = 64
CUT = 6.0

NP = 10240
EP = 163840
NWORK = 32
CHUNK = 64
NCHUNK = EP // (NWORK * CHUNK)
EDGES_PER_W = EP // NWORK
K0 = 90
K1 = 70
OFF0 = 16 * K0 * CHUNK
ROWS_PER_TILE = NP // 16

BN_A = 1024
BE_B = 2000
BN_2 = 1000


def _stage_a_body(disc_ref, emb_ref, wn_ref, bn_ref, w1_ref, w2_ref,
                  a_ref, b_ref):
    disc = disc_ref[...]
    vocab_ids = lax.broadcasted_iota(jnp.int32, (1, 64), 1)
    onehot = (disc == vocab_ids).astype(jnp.float32)
    x = jnp.dot(onehot, emb_ref[...], preferred_element_type=jnp.float32, precision=lax.Precision.HIGHEST)
    h = jnp.maximum(
        jnp.dot(x, wn_ref[...], preferred_element_type=jnp.float32)
        + bn_ref[...], 0.0)
    a_ref[...] = jnp.dot(h, w1_ref[...], preferred_element_type=jnp.float32)
    b_ref[...] = jnp.dot(h, w2_ref[...], preferred_element_type=jnp.float32)


def _stage_a(disc_p, emb_node, w_node, b_node, w1, w2):
    grid = NP // BN_A
    return pl.pallas_call(
        _stage_a_body,
        grid=(grid,),
        in_specs=[
            pl.BlockSpec((BN_A, 1), lambda i: (i, 0)),
            pl.BlockSpec((64, HID), lambda i: (0, 0)),
            pl.BlockSpec((HID, HID), lambda i: (0, 0)),
            pl.BlockSpec((1, HID), lambda i: (0, 0)),
            pl.BlockSpec((HID, HID), lambda i: (0, 0)),
            pl.BlockSpec((HID, HID), lambda i: (0, 0)),
        ],
        out_specs=[
            pl.BlockSpec((BN_A, HID), lambda i: (i, 0)),
            pl.BlockSpec((BN_A, HID), lambda i: (i, 0)),
        ],
        out_shape=[
            jax.ShapeDtypeStruct((NP, HID), jnp.float32),
            jax.ShapeDtypeStruct((NP, HID), jnp.float32),
        ],
    )(disc_p, emb_node, w_node, b_node, w1, w2)


def _stage_b_body(d_ref, we_ref, be_ref, embd_ref, wd_ref, bd_ref,
                  w3_ref, bagg_ref, c_ref):
    d = d_ref[...]
    centers = (lax.broadcasted_iota(jnp.int32, (1, RBF), 1).astype(jnp.float32)
               * (CUT / (RBF - 1)))
    diff = d - centers
    r = d * (1.0 / CUT)
    r2 = r * r
    r3 = r2 * r
    r4 = r2 * r2
    r5 = r4 * r
    env = jnp.clip(1.0 - 6.0 * r5 + 15.0 * r4 - 10.0 * r3, 0.0, 1.0)
    rbf = jnp.exp(-10.0 * diff * diff) * env
    eh_rbf = jnp.maximum(
        jnp.dot(rbf, we_ref[...], preferred_element_type=jnp.float32)
        + be_ref[...], 0.0)
    t8 = jnp.maximum(
        jnp.dot(embd_ref[...], wd_ref[...],
                preferred_element_type=jnp.float32) + bd_ref[...], 0.0)
    dist_idx = jnp.clip(d, 1.0, 4.99999).astype(jnp.int32) - 1
    slot_ids = lax.broadcasted_iota(jnp.int32, (1, 8), 1)
    oh = (dist_idx == slot_ids).astype(jnp.float32)
    eh_emb = jnp.dot(oh, t8, preferred_element_type=jnp.float32,
                     precision=lax.Precision.HIGHEST)
    edge_feat = eh_rbf + eh_emb
    c_ref[...] = (jnp.dot(edge_feat, w3_ref[...],
                          preferred_element_type=jnp.float32)
                  + bagg_ref[...])


def _stage_b(d_p, w_edge, b_edge, emb_dist8, w_dist, b_dist, w3, b_agg):
    grid = E // BE_B
    return pl.pallas_call(
        _stage_b_body,
        grid=(grid,),
        in_specs=[
            pl.BlockSpec((BE_B, 1), lambda i: (i, 0)),
            pl.BlockSpec((RBF, HID), lambda i: (0, 0)),
            pl.BlockSpec((1, HID), lambda i: (0, 0)),
            pl.BlockSpec((8, HID), lambda i: (0, 0)),
            pl.BlockSpec((HID, HID), lambda i: (0, 0)),
            pl.BlockSpec((1, HID), lambda i: (0, 0)),
            pl.BlockSpec((HID, HID), lambda i: (0, 0)),
            pl.BlockSpec((1, HID), lambda i: (0, 0)),
        ],
        out_specs=pl.BlockSpec((BE_B, HID), lambda i: (i, 0)),
        out_shape=jax.ShapeDtypeStruct((EP, HID), jnp.float32),
    )(d_p, w_edge, b_edge, emb_dist8, w_dist, b_dist, w3, b_agg)


def _sc_body(a_hbm, b_hbm, c_hbm, src_hbm, dst_hbm, out_hbm,
             idx_s, idx_d, arows, brows, crows, zrow, accum, sem_a, sem_b):
    cid = lax.axis_index("c")
    sid = lax.axis_index("s")
    nchunks = jnp.where(cid == 0, K0, K1)
    base = jnp.where(cid == 0, sid * (K0 * CHUNK), OFF0 + sid * (K1 * CHUNK))

    def zfill(i, _):
        def zfill_inner(j, _):
            zrow[i, pl.ds(j * 16, 16)] = jnp.zeros((16,), jnp.float32)
            return 0
        return lax.fori_loop(0, HID // 16, zfill_inner, 0)
    lax.fori_loop(0, CHUNK, zfill, 0)
    for k in range(ROWS_PER_TILE // CHUNK):
        pltpu.sync_copy(zrow,
                        accum.at[pl.ds(sid * ROWS_PER_TILE + k * CHUNK, CHUNK)])
    plsc.subcore_barrier()

    def chunk_body(j, _):
        off = base + j * CHUNK
        pltpu.sync_copy(src_hbm.at[pl.ds(off, CHUNK)], idx_s)
        pltpu.sync_copy(dst_hbm.at[pl.ds(off, CHUNK)], idx_d)
        ga = pltpu.async_copy(a_hbm.at[idx_s], arows, sem_a)
        gb = pltpu.async_copy(b_hbm.at[idx_d], brows, sem_b)
        pltpu.sync_copy(c_hbm.at[pl.ds(off, CHUNK)], crows)
        ga.wait()
        gb.wait()

        def row_body(i, _):
            def lane_body(jj, _):
                sl = pl.ds(jj * 16, 16)
                v = arows[i, sl] + brows[i, sl] + crows[i, sl]
                arows[i, sl] = jnp.maximum(v, 0.0)
                return 0
            return lax.fori_loop(0, HID // 16, lane_body, 0)
        lax.fori_loop(0, CHUNK, row_body, 0)

        junk = jnp.full((16,), NP - 1, jnp.int32)
        nv = CHUNK // 16
        iv = [idx_d[pl.ds(t * 16, 16)] for t in range(nv)]
        pos = [lax.iota(jnp.int32, 16) + t * 16 for t in range(nv)]
        one = jnp.full((16,), 1, jnp.int32)
        zero = jnp.zeros((16,), jnp.int32)
        occ = [zero for _ in range(nv)]
        for j in range(CHUNK):
            t0, q = j // 16, j % 16
            b = jnp.full((16,), iv[t0][q], jnp.int32)
            for u in range(t0, nv):
                later = pos[u] > j
                if u == t0 and q == 15:
                    continue
                occ[u] = occ[u] + jnp.where((iv[u] == b) & later, one, zero)
        for u in range(nv):
            occ[u] = jnp.where(iv[u] == junk, zero, occ[u])
        mx = jnp.maximum(jnp.maximum(occ[0], occ[1]),
                         jnp.maximum(occ[2], occ[3]))
        nrounds = mx[0]
        for q in range(1, 16):
            nrounds = jnp.maximum(nrounds, mx[q])
        nrounds = nrounds + 1

        def round_body(r, _):
            rv = jnp.full((16,), r, jnp.int32)
            for u in range(nv):
                m = (occ[u] == rv) & (iv[u] != junk)
                idx_d[pl.ds(u * 16, 16)] = jnp.where(m, iv[u], junk)
            pltpu.sync_copy(arows, accum.at[idx_d], add=True)
            return 0
        lax.fori_loop(0, nrounds, round_body, 0)
        return 0

    lax.fori_loop(0, nchunks, chunk_body, 0)
    plsc.subcore_barrier()
    pltpu.sync_copy(accum.at[pl.ds(sid * ROWS_PER_TILE, ROWS_PER_TILE)],
                    out_hbm.at[cid, pl.ds(sid * ROWS_PER_TILE, ROWS_PER_TILE)])


def _stage_sc(a, b, c, src_p, dst_p):
    mesh = plsc.VectorSubcoreMesh(core_axis_name="c", subcore_axis_name="s")
    fn = pl.kernel(
        _sc_body,
        mesh=mesh,
        out_type=jax.ShapeDtypeStruct((2, NP, HID), jnp.float32),
        scratch_types=[
            pltpu.VMEM((CHUNK,), jnp.int32),
            pltpu.VMEM((CHUNK,), jnp.int32),
            pltpu.VMEM((CHUNK, HID), jnp.float32),
            pltpu.VMEM((CHUNK, HID), jnp.float32),
            pltpu.VMEM((CHUNK, HID), jnp.float32),
            pltpu.VMEM((CHUNK, HID), jnp.float32),
            pltpu.VMEM_SHARED((NP, HID), jnp.float32),
            pltpu.SemaphoreType.DMA,
            pltpu.SemaphoreType.DMA,
        ],
    )
    return fn(a, b, c, src_p, dst_p)


def _stage_2_body(p_ref, nc_ref, wt_ref, wb_ref, bb_ref, wm1_ref, bm1_ref,
                  wm2_ref, bm2_ref, wh_ref, bh_ref, out_ref, gacc):
    i = pl.program_id(0)

    @pl.when(i == 0)
    def _():
        gacc[...] = jnp.zeros_like(gacc)

    atom = p_ref[0] + p_ref[1]
    h2 = jnp.maximum(
        jnp.dot(atom, wt_ref[...], preferred_element_type=jnp.float32)
        + jnp.dot(nc_ref[...], wb_ref[...], preferred_element_type=jnp.float32)
        + bb_ref[...], 0.0)
    gacc[0:1, :] = gacc[0:1, :] + jnp.sum(h2, axis=0, keepdims=True)

    @pl.when(i == (N // BN_2) - 1)
    def _():
        g = gacc[0:1, :]
        m1 = jnp.maximum(
            jnp.dot(g, wm1_ref[...], preferred_element_type=jnp.float32, precision=lax.Precision.HIGHEST)
            + bm1_ref[...], 0.0)
        m2 = jnp.maximum(
            jnp.dot(m1, wm2_ref[...], preferred_element_type=jnp.float32, precision=lax.Precision.HIGHEST)
            + bm2_ref[...], 0.0)
        out_ref[...] = (jnp.dot(m2, wh_ref[...],
                                preferred_element_type=jnp.float32, precision=lax.Precision.HIGHEST)
                        + bh_ref[...])


def _stage_2(parts, ncont, wt, wb, bb, wm1, bm1, wm2, bm2, whead, bhead):
    grid = N // BN_2
    return pl.pallas_call(
        _stage_2_body,
        grid=(grid,),
        in_specs=[
            pl.BlockSpec((2, BN_2, HID), lambda i: (0, i, 0)),
            pl.BlockSpec((BN_2, HID), lambda i: (i, 0)),
            pl.BlockSpec((HID, HID), lambda i: (0, 0)),
            pl.BlockSpec((HID, HID), lambda i: (0, 0)),
            pl.BlockSpec((1, HID), lambda i: (0, 0)),
            pl.BlockSpec((HID, 256), lambda i: (0, 0)),
            pl.BlockSpec((1, 256), lambda i: (0, 0)),
            pl.BlockSpec((256, HID), lambda i: (0, 0)),
            pl.BlockSpec((1, HID), lambda i: (0, 0)),
            pl.BlockSpec((HID, 37), lambda i: (0, 0)),
            pl.BlockSpec((1, 37), lambda i: (0, 0)),
        ],
        out_specs=pl.BlockSpec((1, 37), lambda i: (0, 0)),
        out_shape=jax.ShapeDtypeStruct((1, 37), jnp.float32),
        scratch_shapes=[pltpu.VMEM((8, HID), jnp.float32)],
    )(parts, ncont, wt, wb, bb, wm1, bm1, wm2, bm2, whead, bhead)


def kernel(node_feat_continuous, node_feat_discrete, edge_feat_continuous,
           edge_index, emb_node, W_node, b_node, emb_dist, W_dist, b_dist,
           W_edge, b_edge, W_agg, b_agg, W_b2a, b_b2a, W_m1, b_m1,
           W_m2, b_m2, W_out, b_out, W_sc, b_sc):
    f32 = jnp.float32
    disc = node_feat_discrete.astype(jnp.int32).reshape(N, 1)
    disc_p = jnp.pad(disc, ((0, NP - N), (0, 0)))
    src_p = jnp.pad(edge_index[0].astype(jnp.int32), (0, EP - E),
                    constant_values=NP - 1)
    dst_p = jnp.pad(edge_index[1].astype(jnp.int32), (0, EP - E),
                    constant_values=NP - 1)
    d_p = edge_feat_continuous.astype(f32)

    w1 = W_agg[:HID]
    w2 = W_agg[HID:2 * HID]
    w3 = W_agg[2 * HID:]
    emb_dist8 = jnp.pad(emb_dist.astype(f32), ((0, 3), (0, 0)))
    whead = jnp.concatenate([W_out, W_sc], axis=1)
    bhead = jnp.concatenate([b_out, b_sc]).reshape(1, 37)

    a, b = _stage_a(disc_p, emb_node.astype(f32), W_node, b_node.reshape(1, HID),
                    w1, w2)
    c = _stage_b(d_p, W_edge, b_edge.reshape(1, HID), emb_dist8, W_dist,
                 b_dist.reshape(1, HID), w3, b_agg.reshape(1, HID))
    parts = _stage_sc(a, b, c, src_p, dst_p)
    out37 = _stage_2(parts, node_feat_continuous.astype(f32),
                     W_b2a[:HID], W_b2a[HID:], b_b2a.reshape(1, HID),
                     W_m1, b_m1.reshape(1, 256), W_m2, b_m2.reshape(1, HID),
                     whead, bhead)
    return out37[:, 0:1], out37[:, 1:37]


# speedup vs baseline: 3.1401x; 1.1664x over previous
import functools

import jax
import jax.numpy as jnp
from jax import lax
from jax.experimental import pallas as pl
from jax.experimental.pallas import tpu as pltpu
from jax.experimental.pallas import tpu_sc as plsc

N = 10000
E = 160000
HID = 128
RBF = 64
CUT = 6.0

NP = 10112
EP = 163840
NWORK = 32
CHUNK = 64
NCHUNK = EP // (NWORK * CHUNK)
EDGES_PER_W = EP // NWORK
K0 = 90
K1 = 70
OFF0 = 16 * K0 * CHUNK
ROWS_PER_TILE = NP // 16

BN_A = 1264
BE_B = 2000
BN_2 = 1000


def _stage_a_body(disc_ref, emb_ref, wn_ref, bn_ref, w1_ref, w2_ref,
                  a_ref, b_ref):
    disc = disc_ref[...]
    vocab_ids = lax.broadcasted_iota(jnp.int32, (1, 64), 1)
    onehot = (disc == vocab_ids).astype(jnp.float32)
    x = jnp.dot(onehot, emb_ref[...], preferred_element_type=jnp.float32, precision=lax.Precision.HIGHEST)
    h = jnp.maximum(
        jnp.dot(x, wn_ref[...], preferred_element_type=jnp.float32)
        + bn_ref[...], 0.0)
    a_ref[...] = jnp.dot(h, w1_ref[...], preferred_element_type=jnp.float32)
    b_ref[...] = jnp.dot(h, w2_ref[...], preferred_element_type=jnp.float32)


def _stage_a(disc_p, emb_node, w_node, b_node, w1, w2):
    grid = NP // BN_A
    return pl.pallas_call(
        _stage_a_body,
        grid=(grid,),
        in_specs=[
            pl.BlockSpec((BN_A, 1), lambda i: (i, 0)),
            pl.BlockSpec((64, HID), lambda i: (0, 0)),
            pl.BlockSpec((HID, HID), lambda i: (0, 0)),
            pl.BlockSpec((1, HID), lambda i: (0, 0)),
            pl.BlockSpec((HID, HID), lambda i: (0, 0)),
            pl.BlockSpec((HID, HID), lambda i: (0, 0)),
        ],
        out_specs=[
            pl.BlockSpec((BN_A, HID), lambda i: (i, 0)),
            pl.BlockSpec((BN_A, HID), lambda i: (i, 0)),
        ],
        out_shape=[
            jax.ShapeDtypeStruct((NP, HID), jnp.float32),
            jax.ShapeDtypeStruct((NP, HID), jnp.float32),
        ],
    )(disc_p, emb_node, w_node, b_node, w1, w2)


def _stage_b_body(d_ref, we_ref, be_ref, embd_ref, wd_ref, bd_ref,
                  w3_ref, bagg_ref, c_ref):
    d = d_ref[...]
    centers = (lax.broadcasted_iota(jnp.int32, (1, RBF), 1).astype(jnp.float32)
               * (CUT / (RBF - 1)))
    diff = d - centers
    r = d * (1.0 / CUT)
    r2 = r * r
    r3 = r2 * r
    r4 = r2 * r2
    r5 = r4 * r
    env = jnp.clip(1.0 - 6.0 * r5 + 15.0 * r4 - 10.0 * r3, 0.0, 1.0)
    rbf = jnp.exp(-10.0 * diff * diff) * env
    eh_rbf = jnp.maximum(
        jnp.dot(rbf, we_ref[...], preferred_element_type=jnp.float32)
        + be_ref[...], 0.0)
    t8 = jnp.maximum(
        jnp.dot(embd_ref[...], wd_ref[...],
                preferred_element_type=jnp.float32) + bd_ref[...], 0.0)
    dist_idx = jnp.clip(d, 1.0, 4.99999).astype(jnp.int32) - 1
    slot_ids = lax.broadcasted_iota(jnp.int32, (1, 8), 1)
    oh = (dist_idx == slot_ids).astype(jnp.float32)
    eh_emb = jnp.dot(oh, t8, preferred_element_type=jnp.float32,
                     precision=lax.Precision.HIGHEST)
    edge_feat = eh_rbf + eh_emb
    c_ref[...] = (jnp.dot(edge_feat, w3_ref[...],
                          preferred_element_type=jnp.float32)
                  + bagg_ref[...])


def _stage_b(d_p, w_edge, b_edge, emb_dist8, w_dist, b_dist, w3, b_agg):
    grid = E // BE_B
    return pl.pallas_call(
        _stage_b_body,
        grid=(grid,),
        in_specs=[
            pl.BlockSpec((BE_B, 1), lambda i: (i, 0)),
            pl.BlockSpec((RBF, HID), lambda i: (0, 0)),
            pl.BlockSpec((1, HID), lambda i: (0, 0)),
            pl.BlockSpec((8, HID), lambda i: (0, 0)),
            pl.BlockSpec((HID, HID), lambda i: (0, 0)),
            pl.BlockSpec((1, HID), lambda i: (0, 0)),
            pl.BlockSpec((HID, HID), lambda i: (0, 0)),
            pl.BlockSpec((1, HID), lambda i: (0, 0)),
        ],
        out_specs=pl.BlockSpec((BE_B, HID), lambda i: (i, 0)),
        out_shape=jax.ShapeDtypeStruct((EP, HID), jnp.float32),
    )(d_p, w_edge, b_edge, emb_dist8, w_dist, b_dist, w3, b_agg)


def _sc_body(a_hbm, b_hbm, c_hbm, src_hbm, dst_hbm, out_hbm,
             idx_s0, idx_d0, idx_s1, idx_d1, arows0, brows0, crows0,
             arows1, brows1, crows1, accum,
             sem_a0, sem_b0, sem_c0, sem_a1, sem_b1, sem_c1):
    cid = lax.axis_index("c")
    sid = lax.axis_index("s")
    nchunks = jnp.where(cid == 0, K0, K1)
    base = jnp.where(cid == 0, sid * (K0 * CHUNK), OFF0 + sid * (K1 * CHUNK))

    def zfill(i, _):
        def zfill_inner(j, _):
            arows0[i, pl.ds(j * 16, 16)] = jnp.zeros((16,), jnp.float32)
            return 0
        return lax.fori_loop(0, HID // 16, zfill_inner, 0)
    lax.fori_loop(0, CHUNK, zfill, 0)
    rbase = sid * ROWS_PER_TILE
    for k in range(ROWS_PER_TILE // CHUNK):
        pltpu.sync_copy(arows0, accum.at[pl.ds(rbase + k * CHUNK, CHUNK)])
    rem_rows = ROWS_PER_TILE % CHUNK
    if rem_rows:
        pltpu.sync_copy(
            arows0.at[pl.ds(0, rem_rows)],
            accum.at[pl.ds(rbase + (ROWS_PER_TILE // CHUNK) * CHUNK,
                           rem_rows)])
    plsc.subcore_barrier()

    def prefetch(j, idx_s, idx_d, arows, brows, crows, sem_a, sem_b, sem_c):
        off = base + j * CHUNK
        pltpu.sync_copy(src_hbm.at[pl.ds(off, CHUNK)], idx_s)
        pltpu.sync_copy(dst_hbm.at[pl.ds(off, CHUNK)], idx_d)
        ga = pltpu.async_copy(a_hbm.at[idx_s], arows, sem_a)
        gb = pltpu.async_copy(b_hbm.at[idx_d], brows, sem_b)
        gc = pltpu.async_copy(c_hbm.at[pl.ds(off, CHUNK)], crows, sem_c)
        return ga, gb, gc

    def process(idx_d, arows, brows, crows, sem_a, sem_b, sem_c):
        pltpu.make_async_copy(a_hbm.at[idx_d], arows, sem_a).wait()
        pltpu.make_async_copy(b_hbm.at[idx_d], brows, sem_b).wait()
        pltpu.make_async_copy(c_hbm.at[pl.ds(0, CHUNK)], crows, sem_c).wait()

        def row_body(i, _):
            def lane_body(jj, _):
                sl = pl.ds(jj * 16, 16)
                v = arows[i, sl] + brows[i, sl] + crows[i, sl]
                arows[i, sl] = jnp.maximum(v, 0.0)
                return 0
            return lax.fori_loop(0, HID // 16, lane_body, 0)
        lax.fori_loop(0, CHUNK, row_body, 0)

        junk = jnp.full((16,), NP - 1, jnp.int32)
        nv = CHUNK // 16
        iv = [idx_d[pl.ds(t * 16, 16)] for t in range(nv)]
        pos = [lax.iota(jnp.int32, 16) + t * 16 for t in range(nv)]
        one = jnp.full((16,), 1, jnp.int32)
        zero = jnp.zeros((16,), jnp.int32)
        occ = [zero for _ in range(nv)]
        for j in range(CHUNK):
            t0, q = j // 16, j % 16
            b = jnp.full((16,), iv[t0][q], jnp.int32)
            for u in range(t0, nv):
                later = pos[u] > j
                if u == t0 and q == 15:
                    continue
                occ[u] = occ[u] + jnp.where((iv[u] == b) & later, one, zero)
        for u in range(nv):
            occ[u] = jnp.where(iv[u] == junk, zero, occ[u])
        mx = jnp.maximum(jnp.maximum(occ[0], occ[1]),
                         jnp.maximum(occ[2], occ[3]))
        nrounds = mx[0]
        for q in range(1, 16):
            nrounds = jnp.maximum(nrounds, mx[q])
        nrounds = nrounds + 1

        def round_body(r, _):
            rv = jnp.full((16,), r, jnp.int32)
            for u in range(nv):
                m = (occ[u] == rv) & (iv[u] != junk)
                idx_d[pl.ds(u * 16, 16)] = jnp.where(m, iv[u], junk)
            pltpu.sync_copy(arows, accum.at[idx_d], add=True)
            return 0
        lax.fori_loop(0, nrounds, round_body, 0)

    set0 = (idx_s0, idx_d0, arows0, brows0, crows0, sem_a0, sem_b0, sem_c0)
    set1 = (idx_s1, idx_d1, arows1, brows1, crows1, sem_a1, sem_b1, sem_c1)
    prefetch(0, *set0)

    def pair_body(g, _):
        j0 = 2 * g
        prefetch(j0 + 1, *set1)
        process(*set0[1:])
        prefetch(jnp.minimum(j0 + 2, nchunks - 2), *set0)
        process(*set1[1:])
        return 0
    lax.fori_loop(0, nchunks // 2, pair_body, 0)
    pltpu.make_async_copy(a_hbm.at[idx_s0], arows0, sem_a0).wait()
    pltpu.make_async_copy(b_hbm.at[idx_d0], brows0, sem_b0).wait()
    pltpu.make_async_copy(c_hbm.at[pl.ds(0, CHUNK)], crows0, sem_c0).wait()

    plsc.subcore_barrier()
    pltpu.sync_copy(accum.at[pl.ds(rbase, ROWS_PER_TILE)],
                    out_hbm.at[cid, pl.ds(rbase, ROWS_PER_TILE)])


def _stage_sc(a, b, c, src_p, dst_p):
    mesh = plsc.VectorSubcoreMesh(core_axis_name="c", subcore_axis_name="s")
    fn = pl.kernel(
        _sc_body,
        mesh=mesh,
        out_type=jax.ShapeDtypeStruct((2, NP, HID), jnp.float32),
        scratch_types=(
            [pltpu.VMEM((CHUNK,), jnp.int32)] * 4
            + [pltpu.VMEM((CHUNK, HID), jnp.float32)] * 6
            + [pltpu.VMEM_SHARED((NP, HID), jnp.float32)]
            + [pltpu.SemaphoreType.DMA] * 6
        ),
    )
    return fn(a, b, c, src_p, dst_p)


def _stage_2_body(p_ref, nc_ref, wt_ref, wb_ref, bb_ref, wm1_ref, bm1_ref,
                  wm2_ref, bm2_ref, wh_ref, bh_ref, out_ref, gacc):
    i = pl.program_id(0)

    @pl.when(i == 0)
    def _():
        gacc[...] = jnp.zeros_like(gacc)

    atom = p_ref[0] + p_ref[1]
    h2 = jnp.maximum(
        jnp.dot(atom, wt_ref[...], preferred_element_type=jnp.float32)
        + jnp.dot(nc_ref[...], wb_ref[...], preferred_element_type=jnp.float32)
        + bb_ref[...], 0.0)
    gacc[0:1, :] = gacc[0:1, :] + jnp.sum(h2, axis=0, keepdims=True)

    @pl.when(i == (N // BN_2) - 1)
    def _():
        g = gacc[0:1, :]
        m1 = jnp.maximum(
            jnp.dot(g, wm1_ref[...], preferred_element_type=jnp.float32, precision=lax.Precision.HIGHEST)
            + bm1_ref[...], 0.0)
        m2 = jnp.maximum(
            jnp.dot(m1, wm2_ref[...], preferred_element_type=jnp.float32, precision=lax.Precision.HIGHEST)
            + bm2_ref[...], 0.0)
        out_ref[...] = (jnp.dot(m2, wh_ref[...],
                                preferred_element_type=jnp.float32, precision=lax.Precision.HIGHEST)
                        + bh_ref[...])


def _stage_2(parts, ncont, wt, wb, bb, wm1, bm1, wm2, bm2, whead, bhead):
    grid = N // BN_2
    return pl.pallas_call(
        _stage_2_body,
        grid=(grid,),
        in_specs=[
            pl.BlockSpec((2, BN_2, HID), lambda i: (0, i, 0)),
            pl.BlockSpec((BN_2, HID), lambda i: (i, 0)),
            pl.BlockSpec((HID, HID), lambda i: (0, 0)),
            pl.BlockSpec((HID, HID), lambda i: (0, 0)),
            pl.BlockSpec((1, HID), lambda i: (0, 0)),
            pl.BlockSpec((HID, 256), lambda i: (0, 0)),
            pl.BlockSpec((1, 256), lambda i: (0, 0)),
            pl.BlockSpec((256, HID), lambda i: (0, 0)),
            pl.BlockSpec((1, HID), lambda i: (0, 0)),
            pl.BlockSpec((HID, 37), lambda i: (0, 0)),
            pl.BlockSpec((1, 37), lambda i: (0, 0)),
        ],
        out_specs=pl.BlockSpec((1, 37), lambda i: (0, 0)),
        out_shape=jax.ShapeDtypeStruct((1, 37), jnp.float32),
        scratch_shapes=[pltpu.VMEM((8, HID), jnp.float32)],
    )(parts, ncont, wt, wb, bb, wm1, bm1, wm2, bm2, whead, bhead)


def kernel(node_feat_continuous, node_feat_discrete, edge_feat_continuous,
           edge_index, emb_node, W_node, b_node, emb_dist, W_dist, b_dist,
           W_edge, b_edge, W_agg, b_agg, W_b2a, b_b2a, W_m1, b_m1,
           W_m2, b_m2, W_out, b_out, W_sc, b_sc):
    f32 = jnp.float32
    disc = node_feat_discrete.astype(jnp.int32).reshape(N, 1)
    disc_p = jnp.pad(disc, ((0, NP - N), (0, 0)))
    src_p = jnp.pad(edge_index[0].astype(jnp.int32), (0, EP - E),
                    constant_values=NP - 1)
    dst_p = jnp.pad(edge_index[1].astype(jnp.int32), (0, EP - E),
                    constant_values=NP - 1)
    d_p = edge_feat_continuous.astype(f32)

    w1 = W_agg[:HID]
    w2 = W_agg[HID:2 * HID]
    w3 = W_agg[2 * HID:]
    emb_dist8 = jnp.pad(emb_dist.astype(f32), ((0, 3), (0, 0)))
    whead = jnp.concatenate([W_out, W_sc], axis=1)
    bhead = jnp.concatenate([b_out, b_sc]).reshape(1, 37)

    a, b = _stage_a(disc_p, emb_node.astype(f32), W_node, b_node.reshape(1, HID),
                    w1, w2)
    c = _stage_b(d_p, W_edge, b_edge.reshape(1, HID), emb_dist8, W_dist,
                 b_dist.reshape(1, HID), w3, b_agg.reshape(1, HID))
    parts = _stage_sc(a, b, c, src_p, dst_p)
    out37 = _stage_2(parts, node_feat_continuous.astype(f32),
                     W_b2a[:HID], W_b2a[HID:], b_b2a.reshape(1, HID),
                     W_m1, b_m1.reshape(1, 256), W_m2, b_m2.reshape(1, HID),
                     whead, bhead)
    return out37[:, 0:1], out37[:, 1:37]
